# Initial kernel scaffold; baseline (speedup 1.0000x reference)
import jax
import jax.numpy as jnp
from jax.experimental import pallas as pl


def kernel(x_user, x_item, edge_index_u2i, edge_index_i2u, edge_index_u2u, W):
    raise NotImplementedError("write your pallas kernel here")


# speedup vs baseline: 2.5084x; 2.5084x over previous
import jax
import jax.numpy as jnp
from jax import lax
from jax.experimental import pallas as pl
from jax.experimental.pallas import tpu as pltpu
from jax.experimental.pallas import tpu_sc as plsc

N = 10000
D = 256
E = 160000
HD = 128
NPAD = 10240
NS = 16
NC = 2
CHUNK = 64
NCH = 160
NBLK = 32
CBLK = 16
EPT = NCH * CHUNK
EPAD = NS * EPT
ZB = 16
ZBC = 64
TILE = 512


def _agg_body(x_cat, src_r, dst_r, z64,
              out,
              acc, src_v, dst_v, rows, z_v, sem_a, sem_b):
    c = lax.axis_index("c")
    s = lax.axis_index("s")
    pltpu.sync_copy(z64, z_v)

    zr = NPAD // NS
    for i in range(zr // ZB):
        pltpu.sync_copy(z_v, acc.at[pl.ds(s * zr + i * ZB, ZB)])
    plsc.subcore_barrier()

    def _gather_start(ci, buf, sem):
        pltpu.async_copy(x_cat.at[src_v.at[ci]], buf, sem)

    def _gather_wait(buf, sem):
        pltpu.make_async_copy(x_cat.at[src_v.at[0]], buf, sem).wait()

    def _scatter(ci, buf):
        pltpu.sync_copy(buf, acc.at[dst_v.at[ci]], add=True)

    for b in range(NCH // NBLK):
        pltpu.sync_copy(src_r.at[c, s, pl.ds(b * NBLK, NBLK)], src_v)
        pltpu.sync_copy(dst_r.at[s, pl.ds(b * NBLK, NBLK)], dst_v)
        _gather_start(0, rows.at[0], sem_a)

        def _loop(cb, carry):
            c0 = cb * 2
            _gather_start(c0 + 1, rows.at[1], sem_b)
            _gather_wait(rows.at[0], sem_a)
            _scatter(c0, rows.at[0])

            @pl.when(cb < NBLK // 2 - 1)
            def _():
                _gather_start(c0 + 2, rows.at[0], sem_a)

            _gather_wait(rows.at[1], sem_b)
            _scatter(c0 + 1, rows.at[1])
            return carry

        lax.fori_loop(0, NBLK // 2, _loop, 0)

    plsc.subcore_barrier()

    wr = NPAD // NS

    @pl.when(c == 0)
    def _():
        pltpu.sync_copy(acc.at[pl.ds(s * wr, wr)], out.at[0, pl.ds(s * wr, wr)])

    @pl.when(c == 1)
    def _():
        pltpu.sync_copy(acc.at[pl.ds(s * wr, wr)], out.at[1, pl.ds(s * wr, wr)])


def _make_agg():
    return pl.kernel(
        _agg_body,
        out_type=jax.ShapeDtypeStruct((2, NPAD, HD), jnp.float32),
        mesh=plsc.VectorSubcoreMesh(core_axis_name="c", subcore_axis_name="s"),
        scratch_types=[
            pltpu.VMEM_SHARED((NPAD, HD), jnp.float32),
            pltpu.VMEM((NBLK, CHUNK), jnp.int32),
            pltpu.VMEM((NBLK, CHUNK), jnp.int32),
            pltpu.VMEM((2, CHUNK, HD), jnp.float32),
            pltpu.VMEM((ZB, HD), jnp.float32),
            pltpu.SemaphoreType.DMA,
            pltpu.SemaphoreType.DMA,
        ],
    )


def _cnt_body(dst3, z64, oh3,
              cnts,
              cacc, dst_v, z_v, oh_v):
    c = lax.axis_index("c")
    s = lax.axis_index("s")
    pltpu.sync_copy(z64, z_v)

    zr = NPAD // NS
    for i in range(zr // ZB):
        pltpu.sync_copy(z_v, cacc.at[pl.ds(s * zr + i * ZB, ZB)])
    plsc.subcore_barrier()

    half = NCH // 2
    for t in range(3):
        pltpu.sync_copy(oh3.at[t], oh_v)
        for b in range(half // CBLK):
            pltpu.sync_copy(dst3.at[t, s, pl.ds(c * half + b * CBLK, CBLK)], dst_v)

            def _loop(ci, carry):
                pltpu.sync_copy(oh_v, cacc.at[dst_v.at[ci]], add=True)
                return carry

            lax.fori_loop(0, CBLK, _loop, 0)

    plsc.subcore_barrier()
    wr = NPAD // NS

    @pl.when(c == 0)
    def _():
        pltpu.sync_copy(cacc.at[pl.ds(s * wr, wr)], cnts.at[0, pl.ds(s * wr, wr)])

    @pl.when(c == 1)
    def _():
        pltpu.sync_copy(cacc.at[pl.ds(s * wr, wr)], cnts.at[1, pl.ds(s * wr, wr)])


def _make_cnt():
    return pl.kernel(
        _cnt_body,
        out_type=jax.ShapeDtypeStruct((2, NPAD, HD), jnp.float32),
        mesh=plsc.VectorSubcoreMesh(core_axis_name="c", subcore_axis_name="s"),
        scratch_types=[
            pltpu.VMEM_SHARED((NPAD, HD), jnp.float32),
            pltpu.VMEM((CBLK, CHUNK), jnp.int32),
            pltpu.VMEM((ZB, HD), jnp.float32),
            pltpu.VMEM((CHUNK, HD), jnp.float32),
        ],
    )


def _make_layer_body(split_out):
    def body(xu, xi, si, s1, s2, cnts, wl, *outs):
        def mm2(x2, wmat):
            return (jnp.dot(x2[0], wmat[:HD, :], preferred_element_type=jnp.float32)
                    + jnp.dot(x2[1], wmat[HD:, :], preferred_element_type=jnp.float32))

        def mm2s(x2, scale, wmat):
            return (jnp.dot(x2[0] * scale, wmat[:HD, :], preferred_element_type=jnp.float32)
                    + jnp.dot(x2[1] * scale, wmat[HD:, :], preferred_element_type=jnp.float32))

        inv_i = 1.0 / jnp.maximum(cnts[0, :, 0:1] + cnts[1, :, 0:1], 1.0)
        inv_1 = 1.0 / jnp.maximum(cnts[0, :, 1:2] + cnts[1, :, 1:2], 1.0)
        inv_2 = 1.0 / jnp.maximum(cnts[0, :, 2:3] + cnts[1, :, 2:3], 1.0)

        out_i = mm2(xi, wl[0, 0]) + mm2s(si, inv_i, wl[0, 1])
        out_u = (mm2(xu, wl[1, 0] + wl[2, 0])
                 + mm2s(s1, inv_1, wl[1, 1])
                 + mm2s(s2, inv_2, wl[2, 1]))
        out_i = jnp.maximum(out_i, 0.0)
        out_u = jnp.maximum(out_u, 0.0)

        if split_out:
            xu_o, xi_o = outs
            xu_o[0] = out_u[:, :HD]
            xu_o[1] = out_u[:, HD:]
            xi_o[0] = out_i[:, :HD]
            xi_o[1] = out_i[:, HD:]
        else:
            xu_o, xi_o = outs
            xu_o[...] = out_u
            xi_o[...] = out_i

    return body


def _make_layer(split_out):
    feat = pl.BlockSpec((2, TILE, HD), lambda t: (0, t, 0))
    cnt = pl.BlockSpec((2, TILE, HD), lambda t: (0, t, 0))
    wspec = pl.BlockSpec((3, 2, D, D), lambda t: (0, 0, 0, 0))
    if split_out:
        out_shape = tuple(jax.ShapeDtypeStruct((2, NPAD, HD), jnp.float32) for _ in range(2))
        out_specs = [feat, feat]
    else:
        out_shape = tuple(jax.ShapeDtypeStruct((NPAD, D), jnp.float32) for _ in range(2))
        out_specs = [pl.BlockSpec((TILE, D), lambda t: (t, 0))] * 2
    return pl.pallas_call(
        _make_layer_body(split_out),
        grid=(NPAD // TILE,),
        in_specs=[feat] * 5 + [cnt, wspec],
        out_specs=out_specs,
        out_shape=list(out_shape),
    )


def _split_pad(x):
    lo = jnp.pad(x[:, :HD], ((0, NPAD - N), (0, 0)))
    hi = jnp.pad(x[:, HD:], ((0, NPAD - N), (0, 0)))
    return jnp.stack([lo, hi])


def _prep_edges(ei):
    pad = EPAD - E
    src = jnp.concatenate([ei[0], jnp.zeros((pad,), jnp.int32)])
    dst = jnp.concatenate([ei[1], jnp.full((pad,), NPAD - 1, jnp.int32)])
    src = src.reshape(NS, NCH, CHUNK)
    src2 = jnp.stack([src, src + NPAD])
    return src2, dst.reshape(NS, NCH, CHUNK)


def kernel(x_user, x_item, edge_index_u2i, edge_index_i2u, edge_index_u2u, W):
    xu2 = _split_pad(x_user)
    xi2 = _split_pad(x_item)
    e_u2i = _prep_edges(edge_index_u2i)
    e_i2u = _prep_edges(edge_index_i2u)
    e_u2u = _prep_edges(edge_index_u2u)
    dst3 = jnp.stack([e_u2i[1], e_i2u[1], e_u2u[1]])

    z64 = jnp.zeros((ZB, HD), jnp.float32)
    oh3 = jnp.zeros((3, CHUNK, HD), jnp.float32)
    oh3 = oh3.at[0, :, 0].set(1.0).at[1, :, 1].set(1.0).at[2, :, 2].set(1.0)

    agg = _make_agg()
    cntk = _make_cnt()
    layer_mid = _make_layer(split_out=True)
    layer_last = _make_layer(split_out=False)

    def cat(x2):
        return x2.reshape(2 * NPAD, HD)

    cnts = cntk(dst3, z64, oh3)
    si = agg(cat(xu2), e_u2i[0], e_u2i[1], z64)
    s1 = agg(cat(xi2), e_i2u[0], e_i2u[1], z64)
    s2 = agg(cat(xu2), e_u2u[0], e_u2u[1], z64)
    xu2, xi2 = layer_mid(xu2, xi2, si, s1, s2, cnts, W[0])

    si = agg(cat(xu2), e_u2i[0], e_u2i[1], z64)
    s1 = agg(cat(xi2), e_i2u[0], e_i2u[1], z64)
    s2 = agg(cat(xu2), e_u2u[0], e_u2u[1], z64)
    xu_nat, xi_nat = layer_last(xu2, xi2, si, s1, s2, cnts, W[1])

    return jnp.stack([xu_nat[:N], xi_nat[:N]])


# speedup vs baseline: 2.5860x; 1.0309x over previous
import jax
import jax.numpy as jnp
from jax import lax
from jax.experimental import pallas as pl
from jax.experimental.pallas import tpu as pltpu
from jax.experimental.pallas import tpu_sc as plsc

N = 10000
D = 256
E = 160000
HD = 128
NPAD = 10240
NS = 16
NC = 2
CHUNK = 128
NCH = 80
NBLK = 16
CBLK = 8
EPT = NCH * CHUNK
EPAD = NS * EPT
ZB = 16
ZBC = 64
TILE = 512


def _agg_body(x_cat, src_r, dst_r, z64,
              out,
              acc, src_v, dst_v, rows, z_v, sem_a, sem_b):
    c = lax.axis_index("c")
    s = lax.axis_index("s")
    pltpu.sync_copy(z64, z_v)

    zr = NPAD // NS
    for i in range(zr // ZB):
        pltpu.sync_copy(z_v, acc.at[pl.ds(s * zr + i * ZB, ZB)])
    plsc.subcore_barrier()

    def _gather_start(ci, buf, sem):
        pltpu.async_copy(x_cat.at[src_v.at[ci]], buf, sem)

    def _gather_wait(buf, sem):
        pltpu.make_async_copy(x_cat.at[src_v.at[0]], buf, sem).wait()

    def _scatter(ci, buf):
        pltpu.sync_copy(buf, acc.at[dst_v.at[ci]], add=True)

    for b in range(NCH // NBLK):
        pltpu.sync_copy(src_r.at[c, s, pl.ds(b * NBLK, NBLK)], src_v)
        pltpu.sync_copy(dst_r.at[s, pl.ds(b * NBLK, NBLK)], dst_v)
        _gather_start(0, rows.at[0], sem_a)

        def _loop(cb, carry):
            c0 = cb * 2
            _gather_start(c0 + 1, rows.at[1], sem_b)
            _gather_wait(rows.at[0], sem_a)
            _scatter(c0, rows.at[0])

            @pl.when(cb < NBLK // 2 - 1)
            def _():
                _gather_start(c0 + 2, rows.at[0], sem_a)

            _gather_wait(rows.at[1], sem_b)
            _scatter(c0 + 1, rows.at[1])
            return carry

        lax.fori_loop(0, NBLK // 2, _loop, 0)

    plsc.subcore_barrier()

    wr = NPAD // NS

    @pl.when(c == 0)
    def _():
        pltpu.sync_copy(acc.at[pl.ds(s * wr, wr)], out.at[0, pl.ds(s * wr, wr)])

    @pl.when(c == 1)
    def _():
        pltpu.sync_copy(acc.at[pl.ds(s * wr, wr)], out.at[1, pl.ds(s * wr, wr)])


def _make_agg():
    return pl.kernel(
        _agg_body,
        out_type=jax.ShapeDtypeStruct((2, NPAD, HD), jnp.float32),
        mesh=plsc.VectorSubcoreMesh(core_axis_name="c", subcore_axis_name="s"),
        scratch_types=[
            pltpu.VMEM_SHARED((NPAD, HD), jnp.float32),
            pltpu.VMEM((NBLK, CHUNK), jnp.int32),
            pltpu.VMEM((NBLK, CHUNK), jnp.int32),
            pltpu.VMEM((2, CHUNK, HD), jnp.float32),
            pltpu.VMEM((ZB, HD), jnp.float32),
            pltpu.SemaphoreType.DMA,
            pltpu.SemaphoreType.DMA,
        ],
    )


def _cnt_body(dst3, z64, oh3,
              cnts,
              cacc, dst_v, z_v, oh_v):
    c = lax.axis_index("c")
    s = lax.axis_index("s")
    pltpu.sync_copy(z64, z_v)

    zr = NPAD // NS
    for i in range(zr // ZB):
        pltpu.sync_copy(z_v, cacc.at[pl.ds(s * zr + i * ZB, ZB)])
    plsc.subcore_barrier()

    half = NCH // 2
    for t in range(3):
        pltpu.sync_copy(oh3.at[t], oh_v)
        for b in range(half // CBLK):
            pltpu.sync_copy(dst3.at[t, s, pl.ds(c * half + b * CBLK, CBLK)], dst_v)

            def _loop(ci, carry):
                pltpu.sync_copy(oh_v, cacc.at[dst_v.at[ci]], add=True)
                return carry

            lax.fori_loop(0, CBLK, _loop, 0)

    plsc.subcore_barrier()
    wr = NPAD // NS

    @pl.when(c == 0)
    def _():
        pltpu.sync_copy(cacc.at[pl.ds(s * wr, wr)], cnts.at[0, pl.ds(s * wr, wr)])

    @pl.when(c == 1)
    def _():
        pltpu.sync_copy(cacc.at[pl.ds(s * wr, wr)], cnts.at[1, pl.ds(s * wr, wr)])


def _make_cnt():
    return pl.kernel(
        _cnt_body,
        out_type=jax.ShapeDtypeStruct((2, NPAD, HD), jnp.float32),
        mesh=plsc.VectorSubcoreMesh(core_axis_name="c", subcore_axis_name="s"),
        scratch_types=[
            pltpu.VMEM_SHARED((NPAD, HD), jnp.float32),
            pltpu.VMEM((CBLK, CHUNK), jnp.int32),
            pltpu.VMEM((ZB, HD), jnp.float32),
            pltpu.VMEM((CHUNK, HD), jnp.float32),
        ],
    )


def _make_layer_body(split_out):
    def body(xu, xi, si, s1, s2, cnts, wl, *outs):
        def mm2(x2, wmat):
            return (jnp.dot(x2[0], wmat[:HD, :], preferred_element_type=jnp.float32)
                    + jnp.dot(x2[1], wmat[HD:, :], preferred_element_type=jnp.float32))

        def mm2s(x2, scale, wmat):
            return (jnp.dot(x2[0] * scale, wmat[:HD, :], preferred_element_type=jnp.float32)
                    + jnp.dot(x2[1] * scale, wmat[HD:, :], preferred_element_type=jnp.float32))

        inv_i = 1.0 / jnp.maximum(cnts[0, :, 0:1] + cnts[1, :, 0:1], 1.0)
        inv_1 = 1.0 / jnp.maximum(cnts[0, :, 1:2] + cnts[1, :, 1:2], 1.0)
        inv_2 = 1.0 / jnp.maximum(cnts[0, :, 2:3] + cnts[1, :, 2:3], 1.0)

        out_i = mm2(xi, wl[0, 0]) + mm2s(si, inv_i, wl[0, 1])
        out_u = (mm2(xu, wl[1, 0] + wl[2, 0])
                 + mm2s(s1, inv_1, wl[1, 1])
                 + mm2s(s2, inv_2, wl[2, 1]))
        out_i = jnp.maximum(out_i, 0.0)
        out_u = jnp.maximum(out_u, 0.0)

        if split_out:
            xu_o, xi_o = outs
            xu_o[0] = out_u[:, :HD]
            xu_o[1] = out_u[:, HD:]
            xi_o[0] = out_i[:, :HD]
            xi_o[1] = out_i[:, HD:]
        else:
            xu_o, xi_o = outs
            xu_o[...] = out_u
            xi_o[...] = out_i

    return body


def _make_layer(split_out):
    feat = pl.BlockSpec((2, TILE, HD), lambda t: (0, t, 0))
    cnt = pl.BlockSpec((2, TILE, HD), lambda t: (0, t, 0))
    wspec = pl.BlockSpec((3, 2, D, D), lambda t: (0, 0, 0, 0))
    if split_out:
        out_shape = tuple(jax.ShapeDtypeStruct((2, NPAD, HD), jnp.float32) for _ in range(2))
        out_specs = [feat, feat]
    else:
        out_shape = tuple(jax.ShapeDtypeStruct((NPAD, D), jnp.float32) for _ in range(2))
        out_specs = [pl.BlockSpec((TILE, D), lambda t: (t, 0))] * 2
    return pl.pallas_call(
        _make_layer_body(split_out),
        grid=(NPAD // TILE,),
        in_specs=[feat] * 5 + [cnt, wspec],
        out_specs=out_specs,
        out_shape=list(out_shape),
    )


def _split_pad(x):
    lo = jnp.pad(x[:, :HD], ((0, NPAD - N), (0, 0)))
    hi = jnp.pad(x[:, HD:], ((0, NPAD - N), (0, 0)))
    return jnp.stack([lo, hi])


def _prep_edges(ei):
    pad = EPAD - E
    src = jnp.concatenate([ei[0], jnp.zeros((pad,), jnp.int32)])
    dst = jnp.concatenate([ei[1], jnp.full((pad,), NPAD - 1, jnp.int32)])
    src = src.reshape(NS, NCH, CHUNK)
    src2 = jnp.stack([src, src + NPAD])
    return src2, dst.reshape(NS, NCH, CHUNK)


def kernel(x_user, x_item, edge_index_u2i, edge_index_i2u, edge_index_u2u, W):
    xu2 = _split_pad(x_user)
    xi2 = _split_pad(x_item)
    e_u2i = _prep_edges(edge_index_u2i)
    e_i2u = _prep_edges(edge_index_i2u)
    e_u2u = _prep_edges(edge_index_u2u)
    dst3 = jnp.stack([e_u2i[1], e_i2u[1], e_u2u[1]])

    z64 = jnp.zeros((ZB, HD), jnp.float32)
    oh3 = jnp.zeros((3, CHUNK, HD), jnp.float32)
    oh3 = oh3.at[0, :, 0].set(1.0).at[1, :, 1].set(1.0).at[2, :, 2].set(1.0)

    agg = _make_agg()
    cntk = _make_cnt()
    layer_mid = _make_layer(split_out=True)
    layer_last = _make_layer(split_out=False)

    def cat(x2):
        return x2.reshape(2 * NPAD, HD)

    cnts = cntk(dst3, z64, oh3)
    si = agg(cat(xu2), e_u2i[0], e_u2i[1], z64)
    s1 = agg(cat(xi2), e_i2u[0], e_i2u[1], z64)
    s2 = agg(cat(xu2), e_u2u[0], e_u2u[1], z64)
    xu2, xi2 = layer_mid(xu2, xi2, si, s1, s2, cnts, W[0])

    si = agg(cat(xu2), e_u2i[0], e_u2i[1], z64)
    s1 = agg(cat(xi2), e_i2u[0], e_i2u[1], z64)
    s2 = agg(cat(xu2), e_u2u[0], e_u2u[1], z64)
    xu_nat, xi_nat = layer_last(xu2, xi2, si, s1, s2, cnts, W[1])

    return jnp.stack([xu_nat[:N], xi_nat[:N]])
